# baseline (speedup 1.0000x reference)
import jax
import jax.numpy as jnp
from jax.experimental import pallas as pl
from jax.experimental.pallas import tpu as pltpu

_N = 100000
_CHUNK = 1250
_NCHUNKS = _N // _CHUNK
_NBUF = 8
_LAG = 4


def _body(x_hbm, c_vmem, o_hbm, bufs, fixbuf, sin, sout, sfix):
    ins = [
        pltpu.make_async_copy(
            x_hbm.at[pl.ds(c * _CHUNK, _CHUNK)], bufs.at[c % _NBUF], sin.at[c % _NBUF]
        )
        for c in range(_NCHUNKS)
    ]
    outs = [
        pltpu.make_async_copy(
            bufs.at[c % _NBUF], o_hbm.at[pl.ds(c * _CHUNK, _CHUNK)], sout.at[c % _NBUF]
        )
        for c in range(_NCHUNKS)
    ]
    for c in range(_NCHUNKS + _LAG):
        if c < _NCHUNKS:
            if c >= _NBUF:
                outs[c - _NBUF].wait()
            ins[c].start()
        if c >= _LAG and c - _LAG < _NCHUNKS:
            ins[c - _LAG].wait()
            outs[c - _LAG].start()
    for c in range(_NCHUNKS - _NBUF, _NCHUNKS):
        outs[c].wait()
    fin = pltpu.make_async_copy(x_hbm.at[pl.ds(0, 8)], fixbuf, sfix)
    fin.start()
    fin.wait()
    fixbuf[...] = fixbuf[...] + c_vmem[...]
    fout = pltpu.make_async_copy(fixbuf, o_hbm.at[pl.ds(0, 8)], sfix)
    fout.start()
    fout.wait()


def kernel(x):
    t = jnp.arange(0, 336, 1, dtype=jnp.float32).reshape(7, 6, 8)
    addvals = jnp.concatenate(
        [t[jnp.array([0, 2, 1, 3, 4, 5, 6])], jnp.zeros((1, 6, 8), jnp.float32)], axis=0
    )
    return pl.pallas_call(
        _body,
        in_specs=[
            pl.BlockSpec(memory_space=pl.ANY),
            pl.BlockSpec(memory_space=pltpu.VMEM),
        ],
        out_specs=pl.BlockSpec(memory_space=pl.ANY),
        out_shape=jax.ShapeDtypeStruct((_N, 6, 8), jnp.float32),
        scratch_shapes=[
            pltpu.VMEM((_NBUF, _CHUNK, 6, 8), jnp.float32),
            pltpu.VMEM((8, 6, 8), jnp.float32),
            pltpu.SemaphoreType.DMA((_NBUF,)),
            pltpu.SemaphoreType.DMA((_NBUF,)),
            pltpu.SemaphoreType.DMA,
        ],
    )(x, addvals)
